# Initial kernel scaffold; baseline (speedup 1.0000x reference)
import jax
import jax.numpy as jnp
from jax.experimental import pallas as pl


def kernel(user_ids, item_ids, emb_user, emb_item):
    raise NotImplementedError("write your pallas kernel here")


# speedup vs baseline: 1.5537x; 1.5537x over previous
import functools

import jax
import jax.numpy as jnp
from jax import lax
from jax.experimental import pallas as pl
from jax.experimental.pallas import tpu as pltpu
from jax.experimental.pallas import tpu_sc as plsc

N_USERS = 1000000
N_ITEMS = 1000000
D = 16
B = 16384
L = 50

NC = 2
NS = 16
NW = NC * NS
TOT = B * L
RPW = TOT // NW
CH = 1024
NCH = RPW // CH
GB = 128
NGB = CH // GB
NBLK = CH // 16

_mesh = plsc.VectorSubcoreMesh(
    core_axis_name="c", subcore_axis_name="s", num_cores=NC, num_subcores=NS
)


@functools.partial(
    pl.kernel,
    out_type=jax.ShapeDtypeStruct((TOT,), jnp.float32),
    mesh=_mesh,
    compiler_params=pltpu.CompilerParams(
        needs_layout_passes=False, use_tc_tiling_on_sc=False),
    scratch_types=[
        pltpu.VMEM((CH,), jnp.int32),
        pltpu.VMEM((CH,), jnp.int32),
        pltpu.VMEM((CH, D), jnp.float32),
        pltpu.VMEM((CH, D), jnp.float32),
        pltpu.VMEM((CH,), jnp.float32),
        pltpu.SemaphoreType.DMA,
        pltpu.SemaphoreType.DMA,
    ],
)
def _cml_kernel(uid_hbm, iid_hbm, emb_u_hbm, emb_i_hbm, out_hbm,
                uidx, iidx, eu, ei, outb, sem_u, sem_i):
    wid = lax.axis_index("s") * NC + lax.axis_index("c")
    wbase = wid * RPW

    lane = lax.iota(jnp.int32, 16)
    rots = [(lane + d) & 15 for d in range(D)]

    def chunk_body(c, carry):
        off = wbase + c * CH
        pltpu.sync_copy(uid_hbm.at[pl.ds(off, CH)], uidx)
        pltpu.sync_copy(iid_hbm.at[pl.ds(off, CH)], iidx)

        copies = []
        for j in range(NGB):
            s = j * GB
            copies.append(pltpu.async_copy(
                emb_u_hbm.at[uidx.at[pl.ds(s, GB)]], eu.at[pl.ds(s, GB)], sem_u))
            copies.append(pltpu.async_copy(
                emb_i_hbm.at[iidx.at[pl.ds(s, GB)]], ei.at[pl.ds(s, GB)], sem_i))
        for cp in copies:
            cp.wait()

        def blk_body(b, carry2):
            row = lane + b * 16
            acc = jnp.zeros((16,), jnp.float32)
            for d in range(D):
                vu = plsc.load_gather(eu, [row, rots[d]])
                vi = plsc.load_gather(ei, [row, rots[d]])
                t = vu - vi
                acc = acc + t * t
            outb[pl.ds(b * 16, 16)] = -acc
            return carry2

        lax.fori_loop(0, NBLK, blk_body, 0, unroll=False)
        pltpu.sync_copy(outb, out_hbm.at[pl.ds(off, CH)])
        return carry

    lax.fori_loop(0, NCH, chunk_body, 0, unroll=False)


def kernel(user_ids, item_ids, emb_user, emb_item):
    uid = user_ids.reshape(-1).astype(jnp.int32)
    iid = item_ids.reshape(-1).astype(jnp.int32)
    out = _cml_kernel(uid, iid, emb_user, emb_item)
    return out.reshape(B, L)
